# Initial kernel scaffold; baseline (speedup 1.0000x reference)
import jax
import jax.numpy as jnp
from jax.experimental import pallas as pl


def kernel(x, ei0, ev0, ei2, ev2, W1, b1, W2, b2, Wf1, bf1, Wf2, bf2):
    raise NotImplementedError("write your pallas kernel here")


# speedup vs baseline: 1.0120x; 1.0120x over previous
import jax
import jax.numpy as jnp
from jax.experimental import pallas as pl

V0 = 4096
V1 = 1024
B = 16
K1 = 5
K2 = 5
F1 = 32
F2 = 64
FC1F = 512
FC2F = 10


def _spmm_xla(ei, ev, x, n):
    return jnp.zeros((n, x.shape[1]), x.dtype).at[ei[1]].add(ev[:, None] * x[ei[0]])


def _p1_body(t_ref, p_ref, b_ref, out_ref):
    G = jnp.dot(t_ref[...], p_ref[...], preferred_element_type=jnp.float32)
    G = jnp.maximum(G + b_ref[...], 0.0)
    out_ref[...] = G.reshape(V1, 4, B * F1).max(axis=1)


def _p2a_body(u2_ref, w2_ref, b2_ref, out_ref):
    acc = jnp.broadcast_to(b2_ref[...], (V1 * B, F2))
    for k in range(K2):
        acc = acc + jnp.dot(u2_ref[k], w2_ref[k],
                            preferred_element_type=jnp.float32)
    r = jnp.maximum(acc, 0.0)
    out_ref[...] = r.reshape(V1 // 4, 4, B, F2).max(axis=1).reshape(
        (V1 // 4) * B, F2)


def _p2b_body(a_ref, wf1_ref, bf1_ref, wf2_ref, bf2_ref, out_ref):
    h1 = jax.lax.dot_general(a_ref[...], wf1_ref[...], (((1,), (1,)), ((), ())),
                             preferred_element_type=jnp.float32)
    h1 = jnp.maximum(h1 + bf1_ref[...], 0.0)
    out = jax.lax.dot_general(h1, wf2_ref[...], (((1,), (1,)), ((), ())),
                              preferred_element_type=jnp.float32)
    out_ref[...] = out + bf2_ref[...]


_p1_call = pl.pallas_call(
    _p1_body, out_shape=jax.ShapeDtypeStruct((V1, B * F1), jnp.float32))
_p2a_call = pl.pallas_call(
    _p2a_body, out_shape=jax.ShapeDtypeStruct(((V1 // 4) * B, F2), jnp.float32))
_p2b_call = pl.pallas_call(
    _p2b_body, out_shape=jax.ShapeDtypeStruct((B, FC2F), jnp.float32))


def kernel(x, ei0, ev0, ei2, ev2, W1, b1, W2, b2, Wf1, bf1, Wf2, bf2):
    x0 = x[:, 0, :].T
    xs = [x0]
    xs.append(_spmm_xla(ei0, ev0, x0, V0))
    for _ in range(2, K1):
        xs.append(2.0 * _spmm_xla(ei0, ev0, xs[-1], V0) - xs[-2])
    T = jnp.stack(xs, axis=1).reshape(V0, K1 * B)

    P = jnp.einsum('ck,bd->kbdc', W1, jnp.eye(B, dtype=W1.dtype))
    P = P.reshape(K1 * B, B * F1)
    b1big = jnp.tile(b1, B).reshape(1, B * F1)

    y0 = _p1_call(T, P, b1big)

    ys = [y0]
    ys.append(_spmm_xla(ei2, ev2, y0, V1))
    for _ in range(2, K2):
        ys.append(2.0 * _spmm_xla(ei2, ev2, ys[-1], V1) - ys[-2])
    U2 = jnp.stack(ys, 0).reshape(K2, V1 * B, F1)

    W2kT = jnp.transpose(W2.reshape(F2, F1, K2), (2, 1, 0))
    b2big = b2.reshape(1, F2)

    r4 = _p2a_call(U2, W2kT, b2big)

    A = jnp.transpose(r4.reshape(V1 // 4, B, F2), (1, 2, 0)).reshape(B, -1)

    return _p2b_call(A, Wf1, bf1.reshape(1, FC1F), Wf2, bf2.reshape(1, FC2F))


# speedup vs baseline: 11.9873x; 11.8450x over previous
import functools

import jax
import jax.numpy as jnp
from jax import lax
from jax.experimental import pallas as pl
from jax.experimental.pallas import tpu as pltpu
from jax.experimental.pallas import tpu_sc as plsc

V0 = 4096
V1 = 1024
B = 16
K1 = 5
K2 = 5
F1 = 32
F2 = 64
FC1F = 512
FC2F = 10
E0 = V0 * 16
E2 = V1 * 16

NC = 2
NS = 16
LANES = 16
CH = 128

EPW0 = E0 // (NC * NS)
NCH0 = EPW0 // CH
RPT0 = V0 // NS

EPW2 = E2 // NS
NCH2 = EPW2 // CH
DRT = (V1 * V1 // LANES) // NS


def _sc_spmm1_body(x_hbm, src_hbm, dst_hbm, ev_hbm, out_hbm,
                   idx_s, idx_d, ev_v, rows_v, acc_sh, sem):
    c = lax.axis_index("c")
    s = lax.axis_index("s")
    w = c * NS + s

    def _z(i, _):
        rows_v[i] = jnp.zeros((LANES,), jnp.float32)
        return 0
    lax.fori_loop(0, RPT0, _z, 0)
    pltpu.sync_copy(rows_v.at[pl.ds(0, RPT0)],
                    acc_sh.at[pl.ds(s * RPT0, RPT0)])

    pltpu.sync_copy(src_hbm.at[pl.ds(w * EPW0, EPW0)], idx_s)
    pltpu.sync_copy(dst_hbm.at[pl.ds(w * NCH0, NCH0)], idx_d)
    pltpu.sync_copy(ev_hbm.at[pl.ds(w * EPW0, EPW0)], ev_v)

    plsc.subcore_barrier()

    descs = []
    for j in range(NCH0):
        descs.append(pltpu.async_copy(
            x_hbm.at[idx_s.at[pl.ds(j * CH, CH)]],
            rows_v.at[pl.ds(j * CH, CH)], sem))
    for d in descs:
        d.wait()

    def _m(g, _):
        base = g * LANES
        ev16 = ev_v[pl.ds(base, LANES)]
        for t in range(LANES):
            rows_v[base + t] = rows_v[base + t] * ev16[t]
        return 0
    lax.fori_loop(0, EPW0 // LANES, _m, 0)

    for j in range(NCH0):
        pltpu.sync_copy(rows_v.at[pl.ds(j * CH, CH)],
                        acc_sh.at[idx_d.at[j]], add=True)

    plsc.subcore_barrier()

    pltpu.sync_copy(acc_sh.at[pl.ds(s * RPT0, RPT0)],
                    out_hbm.at[c].at[pl.ds(s * RPT0, RPT0)])


def _sc_densify_body(src_hbm, dst_hbm, ev_hbm, out_hbm,
                     idx_s, idx_d, ev_v, sidx, rows_v, ld_sh, sem):
    c = lax.axis_index("c")
    s = lax.axis_index("s")

    @pl.when(c == 0)
    def _():
        def _z(i, _):
            rows_v[i] = jnp.zeros((LANES,), jnp.float32)
            return 0
        lax.fori_loop(0, RPT0, _z, 0)
        for r in range(DRT // RPT0):
            pltpu.sync_copy(
                rows_v.at[pl.ds(0, RPT0)],
                ld_sh.at[pl.ds(s * DRT + r * RPT0, RPT0)])

        pltpu.sync_copy(src_hbm.at[pl.ds(s * EPW2, EPW2)], idx_s)
        pltpu.sync_copy(dst_hbm.at[pl.ds(s * EPW2, EPW2)], idx_d)
        pltpu.sync_copy(ev_hbm.at[pl.ds(s * EPW2, EPW2)], ev_v)

        plsc.subcore_barrier()

        lane_ids = lax.iota(jnp.int32, LANES)

        def _mk(g, _):
            base = g * LANES
            sv = idx_s[pl.ds(base, LANES)]
            dv = idx_d[pl.ds(base, LANES)]
            ev16 = ev_v[pl.ds(base, LANES)]
            srow = dv * (V1 // LANES) + (sv >> 4)
            lane = sv & (LANES - 1)
            chunk = g // (CH // LANES)
            off = (g % (CH // LANES)) * LANES
            sidx[chunk, pl.ds(off, LANES)] = srow
            for t in range(LANES):
                rows_v[base + t] = jnp.where(lane_ids == lane[t], ev16[t], 0.0)
            return 0
        lax.fori_loop(0, EPW2 // LANES, _mk, 0)

        for j in range(NCH2):
            pltpu.sync_copy(rows_v.at[pl.ds(j * CH, CH)],
                            ld_sh.at[sidx.at[j]], add=True)

        plsc.subcore_barrier()

        pltpu.sync_copy(ld_sh.at[pl.ds(s * DRT, DRT)],
                        out_hbm.at[pl.ds(s * DRT, DRT)])


@functools.lru_cache(maxsize=1)
def _build_sc_kernels():
    mesh = plsc.VectorSubcoreMesh(core_axis_name="c", subcore_axis_name="s")
    params = pltpu.CompilerParams(use_tc_tiling_on_sc=False)
    spmm1 = pl.kernel(
        _sc_spmm1_body,
        out_type=jax.ShapeDtypeStruct((NC, V0, B), jnp.float32),
        mesh=mesh,
        compiler_params=params,
        scratch_types=[
            pltpu.VMEM((EPW0,), jnp.int32),
            pltpu.VMEM((NCH0, CH), jnp.int32),
            pltpu.VMEM((EPW0,), jnp.float32),
            pltpu.VMEM((EPW0, LANES), jnp.float32),
            pltpu.VMEM_SHARED((V0, B), jnp.float32),
            pltpu.SemaphoreType.DMA,
        ],
    )
    densify = pl.kernel(
        _sc_densify_body,
        out_type=jax.ShapeDtypeStruct((V1 * V1 // LANES, LANES), jnp.float32),
        mesh=mesh,
        compiler_params=params,
        scratch_types=[
            pltpu.VMEM((EPW2,), jnp.int32),
            pltpu.VMEM((EPW2,), jnp.int32),
            pltpu.VMEM((EPW2,), jnp.float32),
            pltpu.VMEM((NCH2, CH), jnp.int32),
            pltpu.VMEM((EPW2, LANES), jnp.float32),
            pltpu.VMEM_SHARED((V1 * V1 // LANES, LANES), jnp.float32),
            pltpu.SemaphoreType.DMA,
        ],
    )
    return spmm1, densify


def _comb_first_body(p_ref, out_ref):
    out_ref[...] = p_ref[0] + p_ref[1]


def _comb_next_body(p_ref, prev_ref, out_ref):
    out_ref[...] = 2.0 * (p_ref[0] + p_ref[1]) - prev_ref[...]


def _p1_body(t_ref, p_ref, b_ref, out_ref):
    G = jnp.dot(t_ref[...], p_ref[...], preferred_element_type=jnp.float32)
    G = jnp.maximum(G + b_ref[...], 0.0)
    out_ref[...] = G.reshape(V1, 4, B * F1).max(axis=1)


def _l2_body(ld_ref, y0_ref, q_ref, b2_ref, out_ref):
    Ld = ld_ref[...]
    z0 = y0_ref[...]
    hi = jax.lax.Precision.HIGHEST
    z1 = jnp.dot(Ld, z0, preferred_element_type=jnp.float32, precision=hi)
    z2 = 2.0 * jnp.dot(Ld, z1, preferred_element_type=jnp.float32,
                       precision=hi) - z0
    z3 = 2.0 * jnp.dot(Ld, z2, preferred_element_type=jnp.float32,
                       precision=hi) - z1
    z4 = 2.0 * jnp.dot(Ld, z3, preferred_element_type=jnp.float32,
                       precision=hi) - z2
    acc = jnp.broadcast_to(b2_ref[...], (V1, B * F2))
    for k, z in enumerate((z0, z1, z2, z3, z4)):
        acc = acc + jnp.dot(z, q_ref[k], preferred_element_type=jnp.float32)
    r = jnp.maximum(acc, 0.0)
    out_ref[...] = r.reshape(V1 // 4, 4, B * F2).max(axis=1)


def _p2b_body(a_ref, wf1_ref, bf1_ref, wf2_ref, bf2_ref, out_ref):
    h1 = jax.lax.dot_general(a_ref[...], wf1_ref[...], (((1,), (1,)), ((), ())),
                             preferred_element_type=jnp.float32)
    h1 = jnp.maximum(h1 + bf1_ref[...], 0.0)
    out = jax.lax.dot_general(h1, wf2_ref[...], (((1,), (1,)), ((), ())),
                              preferred_element_type=jnp.float32)
    out_ref[...] = out + bf2_ref[...]


_comb_first = pl.pallas_call(
    _comb_first_body, out_shape=jax.ShapeDtypeStruct((V0, B), jnp.float32))
_comb_next = pl.pallas_call(
    _comb_next_body, out_shape=jax.ShapeDtypeStruct((V0, B), jnp.float32))
_p1_call = pl.pallas_call(
    _p1_body, out_shape=jax.ShapeDtypeStruct((V1, B * F1), jnp.float32))
_l2_call = pl.pallas_call(
    _l2_body, out_shape=jax.ShapeDtypeStruct((V1 // 4, B * F2), jnp.float32))
_p2b_call = pl.pallas_call(
    _p2b_body, out_shape=jax.ShapeDtypeStruct((B, FC2F), jnp.float32))


def kernel(x, ei0, ev0, ei2, ev2, W1, b1, W2, b2, Wf1, bf1, Wf2, bf2):
    src0 = ei0[0]
    dst0 = ei0[1].reshape(E0 // CH, CH)
    src2 = ei2[0]
    dst2 = ei2[1]

    sc_spmm1, sc_densify = _build_sc_kernels()

    ld = sc_densify(src2, dst2, ev2).reshape(V1, V1)

    x0 = x[:, 0, :].T
    p = sc_spmm1(x0, src0, dst0, ev0)
    x1 = _comb_first(p)
    p = sc_spmm1(x1, src0, dst0, ev0)
    x2 = _comb_next(p, x0)
    p = sc_spmm1(x2, src0, dst0, ev0)
    x3 = _comb_next(p, x1)
    p = sc_spmm1(x3, src0, dst0, ev0)
    x4 = _comb_next(p, x2)

    T = jnp.stack([x0, x1, x2, x3, x4], axis=1).reshape(V0, K1 * B)

    P = jnp.einsum('ck,bd->kbdc', W1, jnp.eye(B, dtype=W1.dtype))
    P = P.reshape(K1 * B, B * F1)
    b1big = jnp.tile(b1, B).reshape(1, B * F1)

    y0 = _p1_call(T, P, b1big)

    W2r = W2.reshape(F2, F1, K2)
    Q = jnp.einsum('fck,bd->kbcdf', W2r, jnp.eye(B, dtype=W2.dtype))
    Q = Q.reshape(K2, B * F1, B * F2)
    b2big = jnp.tile(b2, B).reshape(1, B * F2)

    r4 = _l2_call(ld, y0, Q, b2big)

    A = jnp.transpose(r4.reshape(V1 // 4, B, F2), (1, 2, 0)).reshape(B, -1)

    return _p2b_call(A, Wf1, bf1.reshape(1, FC1F), Wf2, bf2.reshape(1, FC2F))


# speedup vs baseline: 12.1833x; 1.0163x over previous
import functools

import jax
import jax.numpy as jnp
from jax import lax
from jax.experimental import pallas as pl
from jax.experimental.pallas import tpu as pltpu
from jax.experimental.pallas import tpu_sc as plsc

V0 = 4096
V1 = 1024
B = 16
K1 = 5
K2 = 5
F1 = 32
F2 = 64
FC1F = 512
FC2F = 10
E0 = V0 * 16
E2 = V1 * 16

NC = 2
NS = 16
LANES = 16
CH = 128

EPW0 = E0 // (NC * NS)
NCH0 = EPW0 // CH
RPT0 = V0 // NS

EPW2 = E2 // NS
NCH2 = EPW2 // CH
DRT = (V1 * V1 // LANES) // NS


def _sc_spmm1_body(x_hbm, src_hbm, dst_hbm, ev_hbm, out_hbm,
                   idx_s, idx_d, ev_v, rows_v, acc_sh, sem, sem2):
    c = lax.axis_index("c")
    s = lax.axis_index("s")
    w = c * NS + s

    def _z(i, _):
        rows_v[i] = jnp.zeros((LANES,), jnp.float32)
        return 0
    lax.fori_loop(0, RPT0, _z, 0)
    pltpu.sync_copy(rows_v.at[pl.ds(0, RPT0)],
                    acc_sh.at[pl.ds(s * RPT0, RPT0)])

    pltpu.sync_copy(src_hbm.at[pl.ds(w * EPW0, EPW0)], idx_s)
    pltpu.sync_copy(dst_hbm.at[pl.ds(w * NCH0, NCH0)], idx_d)
    pltpu.sync_copy(ev_hbm.at[pl.ds(w * EPW0, EPW0)], ev_v)

    plsc.subcore_barrier()

    descs = []
    for j in range(NCH0):
        descs.append(pltpu.async_copy(
            x_hbm.at[idx_s.at[pl.ds(j * CH, CH)]],
            rows_v.at[pl.ds(j * CH, CH)], sem))
    sdescs = []
    for j in range(NCH0):
        descs[j].wait()

        def _m(g, _, base0=j * CH):
            base = base0 + g * LANES
            ev16 = ev_v[pl.ds(base, LANES)]
            for t in range(LANES):
                rows_v[base + t] = rows_v[base + t] * ev16[t]
            return 0
        lax.fori_loop(0, CH // LANES, _m, 0, unroll=2)
        sdescs.append(pltpu.async_copy(
            rows_v.at[pl.ds(j * CH, CH)],
            acc_sh.at[idx_d.at[j]], sem2, add=True))
    for d in sdescs:
        d.wait()

    plsc.subcore_barrier()

    pltpu.sync_copy(acc_sh.at[pl.ds(s * RPT0, RPT0)],
                    out_hbm.at[c].at[pl.ds(s * RPT0, RPT0)])


def _sc_densify_body(src_hbm, dst_hbm, ev_hbm, out_hbm,
                     idx_s, idx_d, ev_v, sidx, rows_v, ld_sh, sem):
    c = lax.axis_index("c")
    s = lax.axis_index("s")

    @pl.when(c == 0)
    def _():
        def _z(i, _):
            rows_v[i] = jnp.zeros((LANES,), jnp.float32)
            return 0
        lax.fori_loop(0, RPT0, _z, 0)
        for r in range(DRT // RPT0):
            pltpu.sync_copy(
                rows_v.at[pl.ds(0, RPT0)],
                ld_sh.at[pl.ds(s * DRT + r * RPT0, RPT0)])

        pltpu.sync_copy(src_hbm.at[pl.ds(s * EPW2, EPW2)], idx_s)
        pltpu.sync_copy(dst_hbm.at[pl.ds(s * EPW2, EPW2)], idx_d)
        pltpu.sync_copy(ev_hbm.at[pl.ds(s * EPW2, EPW2)], ev_v)

        plsc.subcore_barrier()

        lane_ids = lax.iota(jnp.int32, LANES)

        def _mk(g, _):
            base = g * LANES
            sv = idx_s[pl.ds(base, LANES)]
            dv = idx_d[pl.ds(base, LANES)]
            ev16 = ev_v[pl.ds(base, LANES)]
            srow = dv * (V1 // LANES) + (sv >> 4)
            lane = sv & (LANES - 1)
            chunk = g // (CH // LANES)
            off = (g % (CH // LANES)) * LANES
            sidx[chunk, pl.ds(off, LANES)] = srow
            for t in range(LANES):
                rows_v[base + t] = jnp.where(lane_ids == lane[t], ev16[t], 0.0)
            return 0
        lax.fori_loop(0, EPW2 // LANES, _mk, 0)

        sdescs = []
        for j in range(NCH2):
            sdescs.append(pltpu.async_copy(rows_v.at[pl.ds(j * CH, CH)],
                                           ld_sh.at[sidx.at[j]], sem,
                                           add=True))
        for d in sdescs:
            d.wait()

        plsc.subcore_barrier()

        pltpu.sync_copy(ld_sh.at[pl.ds(s * DRT, DRT)],
                        out_hbm.at[pl.ds(s * DRT, DRT)])


@functools.lru_cache(maxsize=1)
def _build_sc_kernels():
    mesh = plsc.VectorSubcoreMesh(core_axis_name="c", subcore_axis_name="s")
    params = pltpu.CompilerParams(use_tc_tiling_on_sc=False)
    spmm1 = pl.kernel(
        _sc_spmm1_body,
        out_type=jax.ShapeDtypeStruct((NC, V0, B), jnp.float32),
        mesh=mesh,
        compiler_params=params,
        scratch_types=[
            pltpu.VMEM((EPW0,), jnp.int32),
            pltpu.VMEM((NCH0, CH), jnp.int32),
            pltpu.VMEM((EPW0,), jnp.float32),
            pltpu.VMEM((EPW0, LANES), jnp.float32),
            pltpu.VMEM_SHARED((V0, B), jnp.float32),
            pltpu.SemaphoreType.DMA,
            pltpu.SemaphoreType.DMA,
        ],
    )
    densify = pl.kernel(
        _sc_densify_body,
        out_type=jax.ShapeDtypeStruct((V1 * V1 // LANES, LANES), jnp.float32),
        mesh=mesh,
        compiler_params=params,
        scratch_types=[
            pltpu.VMEM((EPW2,), jnp.int32),
            pltpu.VMEM((EPW2,), jnp.int32),
            pltpu.VMEM((EPW2,), jnp.float32),
            pltpu.VMEM((NCH2, CH), jnp.int32),
            pltpu.VMEM((EPW2, LANES), jnp.float32),
            pltpu.VMEM_SHARED((V1 * V1 // LANES, LANES), jnp.float32),
            pltpu.SemaphoreType.DMA,
        ],
    )
    return spmm1, densify


def _comb_first_body(p_ref, out_ref):
    out_ref[...] = p_ref[0] + p_ref[1]


def _comb_next_body(p_ref, prev_ref, out_ref):
    out_ref[...] = 2.0 * (p_ref[0] + p_ref[1]) - prev_ref[...]


def _p1_body(t_ref, p_ref, b_ref, out_ref):
    G = jnp.dot(t_ref[...], p_ref[...], preferred_element_type=jnp.float32)
    G = jnp.maximum(G + b_ref[...], 0.0)
    out_ref[...] = G.reshape(V1, 4, B * F1).max(axis=1)


def _l2_body(ld_ref, y0_ref, q_ref, b2_ref, out_ref):
    Ld = ld_ref[...]
    z0 = y0_ref[...]
    hi = jax.lax.Precision.HIGHEST
    z1 = jnp.dot(Ld, z0, preferred_element_type=jnp.float32, precision=hi)
    z2 = 2.0 * jnp.dot(Ld, z1, preferred_element_type=jnp.float32,
                       precision=hi) - z0
    z3 = 2.0 * jnp.dot(Ld, z2, preferred_element_type=jnp.float32,
                       precision=hi) - z1
    z4 = 2.0 * jnp.dot(Ld, z3, preferred_element_type=jnp.float32,
                       precision=hi) - z2
    acc = jnp.broadcast_to(b2_ref[...], (V1, B * F2))
    for k, z in enumerate((z0, z1, z2, z3, z4)):
        acc = acc + jnp.dot(z, q_ref[k], preferred_element_type=jnp.float32)
    r = jnp.maximum(acc, 0.0)
    out_ref[...] = r.reshape(V1 // 4, 4, B * F2).max(axis=1)


def _p2b_body(a_ref, wf1_ref, bf1_ref, wf2_ref, bf2_ref, out_ref):
    h1 = jax.lax.dot_general(a_ref[...], wf1_ref[...], (((1,), (1,)), ((), ())),
                             preferred_element_type=jnp.float32)
    h1 = jnp.maximum(h1 + bf1_ref[...], 0.0)
    out = jax.lax.dot_general(h1, wf2_ref[...], (((1,), (1,)), ((), ())),
                              preferred_element_type=jnp.float32)
    out_ref[...] = out + bf2_ref[...]


_comb_first = pl.pallas_call(
    _comb_first_body, out_shape=jax.ShapeDtypeStruct((V0, B), jnp.float32))
_comb_next = pl.pallas_call(
    _comb_next_body, out_shape=jax.ShapeDtypeStruct((V0, B), jnp.float32))
_p1_call = pl.pallas_call(
    _p1_body, out_shape=jax.ShapeDtypeStruct((V1, B * F1), jnp.float32))
_l2_call = pl.pallas_call(
    _l2_body, out_shape=jax.ShapeDtypeStruct((V1 // 4, B * F2), jnp.float32))
_p2b_call = pl.pallas_call(
    _p2b_body, out_shape=jax.ShapeDtypeStruct((B, FC2F), jnp.float32))


def kernel(x, ei0, ev0, ei2, ev2, W1, b1, W2, b2, Wf1, bf1, Wf2, bf2):
    src0 = ei0[0]
    dst0 = ei0[1].reshape(E0 // CH, CH)
    src2 = ei2[0]
    dst2 = ei2[1]

    sc_spmm1, sc_densify = _build_sc_kernels()

    ld = sc_densify(src2, dst2, ev2).reshape(V1, V1)

    x0 = x[:, 0, :].T
    p = sc_spmm1(x0, src0, dst0, ev0)
    x1 = _comb_first(p)
    p = sc_spmm1(x1, src0, dst0, ev0)
    x2 = _comb_next(p, x0)
    p = sc_spmm1(x2, src0, dst0, ev0)
    x3 = _comb_next(p, x1)
    p = sc_spmm1(x3, src0, dst0, ev0)
    x4 = _comb_next(p, x2)

    T = jnp.stack([x0, x1, x2, x3, x4], axis=1).reshape(V0, K1 * B)

    P = jnp.einsum('ck,bd->kbdc', W1, jnp.eye(B, dtype=W1.dtype))
    P = P.reshape(K1 * B, B * F1)
    b1big = jnp.tile(b1, B).reshape(1, B * F1)

    y0 = _p1_call(T, P, b1big)

    W2r = W2.reshape(F2, F1, K2)
    Q = jnp.einsum('fck,bd->kbcdf', W2r, jnp.eye(B, dtype=W2.dtype))
    Q = Q.reshape(K2, B * F1, B * F2)
    b2big = jnp.tile(b2, B).reshape(1, B * F2)

    r4 = _l2_call(ld, y0, Q, b2big)

    A = jnp.transpose(r4.reshape(V1 // 4, B, F2), (1, 2, 0)).reshape(B, -1)

    return _p2b_call(A, Wf1, bf1.reshape(1, FC1F), Wf2, bf2.reshape(1, FC2F))


# speedup vs baseline: 12.2118x; 1.0023x over previous
import functools

import jax
import jax.numpy as jnp
from jax import lax
from jax.experimental import pallas as pl
from jax.experimental.pallas import tpu as pltpu
from jax.experimental.pallas import tpu_sc as plsc

V0 = 4096
V1 = 1024
B = 16
K1 = 5
K2 = 5
F1 = 32
F2 = 64
FC1F = 512
FC2F = 10
E0 = V0 * 16
E2 = V1 * 16

NC = 2
NS = 16
LANES = 16
CH = 128

EPW0 = E0 // (NC * NS)
NCH0 = EPW0 // CH
RPT0 = V0 // NS

EPW2 = E2 // NS
NCH2 = EPW2 // CH
DRT = (V1 * V1 // LANES) // NS


def _sc_spmm1_body(x_hbm, src_hbm, dst_hbm, ev_hbm, out_hbm,
                   idx_s, idx_d, ev_v, rows_v, acc_sh, sem, sem2):
    c = lax.axis_index("c")
    s = lax.axis_index("s")
    w = c * NS + s

    def _z(i, _):
        rows_v[i] = jnp.zeros((LANES,), jnp.float32)
        return 0
    lax.fori_loop(0, RPT0, _z, 0)
    pltpu.sync_copy(rows_v.at[pl.ds(0, RPT0)],
                    acc_sh.at[pl.ds(s * RPT0, RPT0)])

    pltpu.sync_copy(src_hbm.at[pl.ds(w * EPW0, EPW0)], idx_s)
    pltpu.sync_copy(dst_hbm.at[pl.ds(w * NCH0, NCH0)], idx_d)
    pltpu.sync_copy(ev_hbm.at[pl.ds(w * EPW0, EPW0)], ev_v)

    plsc.subcore_barrier()

    descs = []
    for j in range(NCH0):
        descs.append(pltpu.async_copy(
            x_hbm.at[idx_s.at[pl.ds(j * CH, CH)]],
            rows_v.at[pl.ds(j * CH, CH)], sem))
    sdescs = []
    for j in range(NCH0):
        descs[j].wait()

        def _m(g, _, base0=j * CH):
            base = base0 + g * LANES
            ev16 = ev_v[pl.ds(base, LANES)]
            for t in range(LANES):
                rows_v[base + t] = rows_v[base + t] * ev16[t]
            return 0
        lax.fori_loop(0, CH // LANES, _m, 0, unroll=2)
        sdescs.append(pltpu.async_copy(
            rows_v.at[pl.ds(j * CH, CH)],
            acc_sh.at[idx_d.at[j]], sem2, add=True))
    for d in sdescs:
        d.wait()

    plsc.subcore_barrier()

    pltpu.sync_copy(acc_sh.at[pl.ds(s * RPT0, RPT0)],
                    out_hbm.at[c].at[pl.ds(s * RPT0, RPT0)])


def _sc_densify_body(src_hbm, dst_hbm, ev_hbm, out_hbm,
                     idx_s, idx_d, ev_v, sidx, rows_v, ld_sh, sem):
    c = lax.axis_index("c")
    s = lax.axis_index("s")

    @pl.when(c == 0)
    def _():
        def _z(i, _):
            rows_v[i] = jnp.zeros((LANES,), jnp.float32)
            return 0
        lax.fori_loop(0, RPT0, _z, 0)
        for r in range(DRT // RPT0):
            pltpu.sync_copy(
                rows_v.at[pl.ds(0, RPT0)],
                ld_sh.at[pl.ds(s * DRT + r * RPT0, RPT0)])

        pltpu.sync_copy(src_hbm.at[pl.ds(s * EPW2, EPW2)], idx_s)
        pltpu.sync_copy(dst_hbm.at[pl.ds(s * EPW2, EPW2)], idx_d)
        pltpu.sync_copy(ev_hbm.at[pl.ds(s * EPW2, EPW2)], ev_v)

        plsc.subcore_barrier()

        lane_ids = lax.iota(jnp.int32, LANES)

        def _mk(g, _):
            base = g * LANES
            sv = idx_s[pl.ds(base, LANES)]
            dv = idx_d[pl.ds(base, LANES)]
            ev16 = ev_v[pl.ds(base, LANES)]
            srow = dv * (V1 // LANES) + (sv >> 4)
            lane = sv & (LANES - 1)
            chunk = g // (CH // LANES)
            off = (g % (CH // LANES)) * LANES
            sidx[chunk, pl.ds(off, LANES)] = srow
            for t in range(LANES):
                rows_v[base + t] = jnp.where(lane_ids == lane[t], ev16[t], 0.0)
            return 0
        lax.fori_loop(0, EPW2 // LANES, _mk, 0)

        sdescs = []
        for j in range(NCH2):
            sdescs.append(pltpu.async_copy(rows_v.at[pl.ds(j * CH, CH)],
                                           ld_sh.at[sidx.at[j]], sem,
                                           add=True))
        for d in sdescs:
            d.wait()

        plsc.subcore_barrier()

        pltpu.sync_copy(ld_sh.at[pl.ds(s * DRT, DRT)],
                        out_hbm.at[pl.ds(s * DRT, DRT)])


@functools.lru_cache(maxsize=1)
def _build_sc_kernels():
    mesh = plsc.VectorSubcoreMesh(core_axis_name="c", subcore_axis_name="s")
    params = pltpu.CompilerParams(use_tc_tiling_on_sc=False)
    spmm1 = pl.kernel(
        _sc_spmm1_body,
        out_type=jax.ShapeDtypeStruct((NC, V0, B), jnp.float32),
        mesh=mesh,
        compiler_params=params,
        scratch_types=[
            pltpu.VMEM((EPW0,), jnp.int32),
            pltpu.VMEM((NCH0, CH), jnp.int32),
            pltpu.VMEM((EPW0,), jnp.float32),
            pltpu.VMEM((EPW0, LANES), jnp.float32),
            pltpu.VMEM_SHARED((V0, B), jnp.float32),
            pltpu.SemaphoreType.DMA,
            pltpu.SemaphoreType.DMA,
        ],
    )
    densify = pl.kernel(
        _sc_densify_body,
        out_type=jax.ShapeDtypeStruct((V1 * V1 // LANES, LANES), jnp.float32),
        mesh=mesh,
        compiler_params=params,
        scratch_types=[
            pltpu.VMEM((EPW2,), jnp.int32),
            pltpu.VMEM((EPW2,), jnp.int32),
            pltpu.VMEM((EPW2,), jnp.float32),
            pltpu.VMEM((NCH2, CH), jnp.int32),
            pltpu.VMEM((EPW2, LANES), jnp.float32),
            pltpu.VMEM_SHARED((V1 * V1 // LANES, LANES), jnp.float32),
            pltpu.SemaphoreType.DMA,
        ],
    )
    return spmm1, densify


def _comb_first_body(p_ref, out_ref):
    out_ref[...] = p_ref[0] + p_ref[1]


def _comb_next_body(p_ref, prev_ref, out_ref):
    out_ref[...] = 2.0 * (p_ref[0] + p_ref[1]) - prev_ref[...]


def _p1_body(t_ref, p_ref, b_ref, out_ref):
    G = jnp.dot(t_ref[...], p_ref[...], preferred_element_type=jnp.float32)
    G = jnp.maximum(G + b_ref[...], 0.0)
    out_ref[...] = G.reshape(V1, 4, B * F1).max(axis=1)


def _l2_body(ld_ref, y0_ref, q_ref, b2_ref, out_ref):
    Ld = ld_ref[...]
    z0 = y0_ref[...]
    hi = jax.lax.Precision.HIGHEST
    z1 = jnp.dot(Ld, z0, preferred_element_type=jnp.float32, precision=hi)
    z2 = 2.0 * jnp.dot(Ld, z1, preferred_element_type=jnp.float32,
                       precision=hi) - z0
    z3 = 2.0 * jnp.dot(Ld, z2, preferred_element_type=jnp.float32) - z1
    z4 = 2.0 * jnp.dot(Ld, z3, preferred_element_type=jnp.float32) - z2
    acc = jnp.broadcast_to(b2_ref[...], (V1, B * F2))
    for k, z in enumerate((z0, z1, z2, z3, z4)):
        acc = acc + jnp.dot(z, q_ref[k], preferred_element_type=jnp.float32)
    r = jnp.maximum(acc, 0.0)
    out_ref[...] = r.reshape(V1 // 4, 4, B * F2).max(axis=1)


def _p2b_body(a_ref, wf1_ref, bf1_ref, wf2_ref, bf2_ref, out_ref):
    h1 = jax.lax.dot_general(a_ref[...], wf1_ref[...], (((1,), (1,)), ((), ())),
                             preferred_element_type=jnp.float32)
    h1 = jnp.maximum(h1 + bf1_ref[...], 0.0)
    out = jax.lax.dot_general(h1, wf2_ref[...], (((1,), (1,)), ((), ())),
                              preferred_element_type=jnp.float32)
    out_ref[...] = out + bf2_ref[...]


_comb_first = pl.pallas_call(
    _comb_first_body, out_shape=jax.ShapeDtypeStruct((V0, B), jnp.float32))
_comb_next = pl.pallas_call(
    _comb_next_body, out_shape=jax.ShapeDtypeStruct((V0, B), jnp.float32))
_p1_call = pl.pallas_call(
    _p1_body, out_shape=jax.ShapeDtypeStruct((V1, B * F1), jnp.float32))
_l2_call = pl.pallas_call(
    _l2_body, out_shape=jax.ShapeDtypeStruct((V1 // 4, B * F2), jnp.float32))
_p2b_call = pl.pallas_call(
    _p2b_body, out_shape=jax.ShapeDtypeStruct((B, FC2F), jnp.float32))


def kernel(x, ei0, ev0, ei2, ev2, W1, b1, W2, b2, Wf1, bf1, Wf2, bf2):
    src0 = ei0[0]
    dst0 = ei0[1].reshape(E0 // CH, CH)
    src2 = ei2[0]
    dst2 = ei2[1]

    sc_spmm1, sc_densify = _build_sc_kernels()

    ld = sc_densify(src2, dst2, ev2).reshape(V1, V1)

    x0 = x[:, 0, :].T
    p = sc_spmm1(x0, src0, dst0, ev0)
    x1 = _comb_first(p)
    p = sc_spmm1(x1, src0, dst0, ev0)
    x2 = _comb_next(p, x0)
    p = sc_spmm1(x2, src0, dst0, ev0)
    x3 = _comb_next(p, x1)
    p = sc_spmm1(x3, src0, dst0, ev0)
    x4 = _comb_next(p, x2)

    T = jnp.stack([x0, x1, x2, x3, x4], axis=1).reshape(V0, K1 * B)

    P = jnp.einsum('ck,bd->kbdc', W1, jnp.eye(B, dtype=W1.dtype))
    P = P.reshape(K1 * B, B * F1)
    b1big = jnp.tile(b1, B).reshape(1, B * F1)

    y0 = _p1_call(T, P, b1big)

    W2kT = jnp.transpose(W2.reshape(F2, F1, K2), (2, 1, 0))
    mask = (jnp.arange(B * F1)[:, None] // F1) == (jnp.arange(B * F2)[None, :] // F2)
    Q = jnp.where(mask[None], jnp.tile(W2kT, (1, B, B)), 0.0)
    b2big = jnp.tile(b2, B).reshape(1, B * F2)

    r4 = _l2_call(ld, y0, Q, b2big)

    A = jnp.transpose(r4.reshape(V1 // 4, B, F2), (1, 2, 0)).reshape(B, -1)

    return _p2b_call(A, Wf1, bf1.reshape(1, FC1F), Wf2, bf2.reshape(1, FC2F))


# speedup vs baseline: 12.9915x; 1.0638x over previous
import functools

import jax
import jax.numpy as jnp
from jax import lax
from jax.experimental import pallas as pl
from jax.experimental.pallas import tpu as pltpu
from jax.experimental.pallas import tpu_sc as plsc

V0 = 4096
V1 = 1024
B = 16
K1 = 5
K2 = 5
F1 = 32
F2 = 64
FC1F = 512
FC2F = 10
E0 = V0 * 16
E2 = V1 * 16

NC = 2
NS = 16
LANES = 16
CH = 128

EPW0 = E0 // (NC * NS)
NCH0 = EPW0 // CH
RPT0 = V0 // NS

EPW2 = E2 // NS
NCH2 = EPW2 // CH
DRT = (V1 * V1 // LANES) // NS


def _sc_spmm1_body(x_hbm, src_hbm, dst_hbm, ev_hbm, out_hbm,
                   idx_s, idx_d, ev_v, rows_v, acc_sh, sem, sem2):
    c = lax.axis_index("c")
    s = lax.axis_index("s")
    w = c * NS + s

    def _z(i, _):
        rows_v[i] = jnp.zeros((LANES,), jnp.float32)
        return 0
    lax.fori_loop(0, RPT0, _z, 0)
    pltpu.sync_copy(rows_v.at[pl.ds(0, RPT0)],
                    acc_sh.at[pl.ds(s * RPT0, RPT0)])

    pltpu.sync_copy(src_hbm.at[pl.ds(w * EPW0, EPW0)], idx_s)
    pltpu.sync_copy(dst_hbm.at[pl.ds(w * NCH0, NCH0)], idx_d)
    pltpu.sync_copy(ev_hbm.at[pl.ds(w * EPW0, EPW0)], ev_v)

    plsc.subcore_barrier()

    descs = []
    for j in range(NCH0):
        descs.append(pltpu.async_copy(
            x_hbm.at[idx_s.at[pl.ds(j * CH, CH)]],
            rows_v.at[pl.ds(j * CH, CH)], sem))
    sdescs = []
    for j in range(NCH0):
        descs[j].wait()

        def _m(g, _, base0=j * CH):
            base = base0 + g * LANES
            ev16 = ev_v[pl.ds(base, LANES)]
            for t in range(LANES):
                rows_v[base + t] = rows_v[base + t] * ev16[t]
            return 0
        lax.fori_loop(0, CH // LANES, _m, 0, unroll=2)
        sdescs.append(pltpu.async_copy(
            rows_v.at[pl.ds(j * CH, CH)],
            acc_sh.at[idx_d.at[j]], sem2, add=True))
    for d in sdescs:
        d.wait()

    plsc.subcore_barrier()

    pltpu.sync_copy(acc_sh.at[pl.ds(s * RPT0, RPT0)],
                    out_hbm.at[c].at[pl.ds(s * RPT0, RPT0)])


def _sc_densify_body(src_hbm, dst_hbm, ev_hbm, out_hbm,
                     idx_s, idx_d, ev_v, sidx, rows_v, ld_sh, sem):
    c = lax.axis_index("c")
    s = lax.axis_index("s")

    @pl.when(c == 0)
    def _():
        def _z(i, _):
            rows_v[i] = jnp.zeros((LANES,), jnp.float32)
            return 0
        lax.fori_loop(0, RPT0, _z, 0)
        for r in range(DRT // RPT0):
            pltpu.sync_copy(
                rows_v.at[pl.ds(0, RPT0)],
                ld_sh.at[pl.ds(s * DRT + r * RPT0, RPT0)])

        pltpu.sync_copy(src_hbm.at[pl.ds(s * EPW2, EPW2)], idx_s)
        pltpu.sync_copy(dst_hbm.at[pl.ds(s * EPW2, EPW2)], idx_d)
        pltpu.sync_copy(ev_hbm.at[pl.ds(s * EPW2, EPW2)], ev_v)

        plsc.subcore_barrier()

        lane_ids = lax.iota(jnp.int32, LANES)

        def _mk(g, _):
            base = g * LANES
            sv = idx_s[pl.ds(base, LANES)]
            dv = idx_d[pl.ds(base, LANES)]
            ev16 = ev_v[pl.ds(base, LANES)]
            srow = dv * (V1 // LANES) + (sv >> 4)
            lane = sv & (LANES - 1)
            chunk = g // (CH // LANES)
            off = (g % (CH // LANES)) * LANES
            sidx[chunk, pl.ds(off, LANES)] = srow
            for t in range(LANES):
                rows_v[base + t] = jnp.where(lane_ids == lane[t], ev16[t], 0.0)
            return 0
        lax.fori_loop(0, EPW2 // LANES, _mk, 0)

        sdescs = []
        for j in range(NCH2):
            sdescs.append(pltpu.async_copy(rows_v.at[pl.ds(j * CH, CH)],
                                           ld_sh.at[sidx.at[j]], sem,
                                           add=True))
        for d in sdescs:
            d.wait()

        plsc.subcore_barrier()

        pltpu.sync_copy(ld_sh.at[pl.ds(s * DRT, DRT)],
                        out_hbm.at[pl.ds(s * DRT, DRT)])


@functools.lru_cache(maxsize=1)
def _build_sc_kernels():
    mesh = plsc.VectorSubcoreMesh(core_axis_name="c", subcore_axis_name="s")
    params = pltpu.CompilerParams(use_tc_tiling_on_sc=False)
    spmm1 = pl.kernel(
        _sc_spmm1_body,
        out_type=jax.ShapeDtypeStruct((NC, V0, B), jnp.float32),
        mesh=mesh,
        compiler_params=params,
        scratch_types=[
            pltpu.VMEM((EPW0,), jnp.int32),
            pltpu.VMEM((NCH0, CH), jnp.int32),
            pltpu.VMEM((EPW0,), jnp.float32),
            pltpu.VMEM((EPW0, LANES), jnp.float32),
            pltpu.VMEM_SHARED((V0, B), jnp.float32),
            pltpu.SemaphoreType.DMA,
            pltpu.SemaphoreType.DMA,
        ],
    )
    densify = pl.kernel(
        _sc_densify_body,
        out_type=jax.ShapeDtypeStruct((V1 * V1 // LANES, LANES), jnp.float32),
        mesh=mesh,
        compiler_params=params,
        scratch_types=[
            pltpu.VMEM((EPW2,), jnp.int32),
            pltpu.VMEM((EPW2,), jnp.int32),
            pltpu.VMEM((EPW2,), jnp.float32),
            pltpu.VMEM((NCH2, CH), jnp.int32),
            pltpu.VMEM((EPW2, LANES), jnp.float32),
            pltpu.VMEM_SHARED((V1 * V1 // LANES, LANES), jnp.float32),
            pltpu.SemaphoreType.DMA,
        ],
    )
    return spmm1, densify


def _comb_first_body(p_ref, out_ref):
    out_ref[...] = p_ref[0] + p_ref[1]


def _comb_next_body(p_ref, prev_ref, out_ref):
    out_ref[...] = 2.0 * (p_ref[0] + p_ref[1]) - prev_ref[...]


def _p1_body(t_ref, p_ref, b_ref, out_ref):
    G = jnp.dot(t_ref[...], p_ref[...], preferred_element_type=jnp.float32)
    G = jnp.maximum(G + b_ref[...], 0.0)
    out_ref[...] = G.reshape(V1, 4, B * F1).max(axis=1)


def _l2a_body(ld_ref, y0_ref, zs_ref):
    Ld = ld_ref[...]
    z0 = y0_ref[...]
    hi = jax.lax.Precision.HIGHEST
    z1 = jnp.dot(Ld, z0, preferred_element_type=jnp.float32, precision=hi)
    z2 = 2.0 * jnp.dot(Ld, z1, preferred_element_type=jnp.float32,
                       precision=hi) - z0
    z3 = 2.0 * jnp.dot(Ld, z2, preferred_element_type=jnp.float32) - z1
    z4 = 2.0 * jnp.dot(Ld, z3, preferred_element_type=jnp.float32) - z2
    zs_ref[0] = z0
    zs_ref[1] = z1
    zs_ref[2] = z2
    zs_ref[3] = z3
    zs_ref[4] = z4


def _l2b_body(zs2_ref, w2_ref, b2_ref, out_ref):
    acc = jnp.broadcast_to(b2_ref[...], (V1 * B, F2))
    for k in range(K2):
        acc = acc + jnp.dot(zs2_ref[k], w2_ref[k],
                            preferred_element_type=jnp.float32)
    r = jnp.maximum(acc, 0.0)
    out_ref[...] = r.reshape(V1 // 4, 4, B, F2).max(axis=1).reshape(
        (V1 // 4) * B, F2)


def _p2b_body(a_ref, wf1_ref, bf1_ref, wf2_ref, bf2_ref, out_ref):
    h1 = jax.lax.dot_general(a_ref[...], wf1_ref[...], (((1,), (1,)), ((), ())),
                             preferred_element_type=jnp.float32)
    h1 = jnp.maximum(h1 + bf1_ref[...], 0.0)
    out = jax.lax.dot_general(h1, wf2_ref[...], (((1,), (1,)), ((), ())),
                              preferred_element_type=jnp.float32)
    out_ref[...] = out + bf2_ref[...]


_comb_first = pl.pallas_call(
    _comb_first_body,
    out_shape=jax.ShapeDtypeStruct((V0 * B // 128, 128), jnp.float32))
_comb_next = pl.pallas_call(
    _comb_next_body,
    out_shape=jax.ShapeDtypeStruct((V0 * B // 128, 128), jnp.float32))
_p1_call = pl.pallas_call(
    _p1_body, out_shape=jax.ShapeDtypeStruct((V1, B * F1), jnp.float32))
_l2a_call = pl.pallas_call(
    _l2a_body, out_shape=jax.ShapeDtypeStruct((K2, V1, B * F1), jnp.float32))
_l2b_call = pl.pallas_call(
    _l2b_body, out_shape=jax.ShapeDtypeStruct(((V1 // 4) * B, F2), jnp.float32))
_p2b_call = pl.pallas_call(
    _p2b_body, out_shape=jax.ShapeDtypeStruct((B, FC2F), jnp.float32))


def kernel(x, ei0, ev0, ei2, ev2, W1, b1, W2, b2, Wf1, bf1, Wf2, bf2):
    src0 = ei0[0]
    dst0 = ei0[1].reshape(E0 // CH, CH)
    src2 = ei2[0]
    dst2 = ei2[1]

    sc_spmm1, sc_densify = _build_sc_kernels()

    ld = sc_densify(src2, dst2, ev2).reshape(V1, V1)

    w128 = (NC, V0 * B // 128, 128)
    x0 = x[:, 0, :].T
    x0w = x0.reshape(w128[1:])
    p = sc_spmm1(x0, src0, dst0, ev0)
    x1w = _comb_first(p.reshape(w128))
    p = sc_spmm1(x1w.reshape(V0, B), src0, dst0, ev0)
    x2w = _comb_next(p.reshape(w128), x0w)
    p = sc_spmm1(x2w.reshape(V0, B), src0, dst0, ev0)
    x3w = _comb_next(p.reshape(w128), x1w)
    p = sc_spmm1(x3w.reshape(V0, B), src0, dst0, ev0)
    x4w = _comb_next(p.reshape(w128), x2w)

    xs = [x0] + [a.reshape(V0, B) for a in (x1w, x2w, x3w, x4w)]
    T = jnp.stack(xs, axis=1).reshape(V0, K1 * B)

    P = jnp.einsum('ck,bd->kbdc', W1, jnp.eye(B, dtype=W1.dtype))
    P = P.reshape(K1 * B, B * F1)
    b1big = jnp.tile(b1, B).reshape(1, B * F1)

    y0 = _p1_call(T, P, b1big)

    W2kT = jnp.transpose(W2.reshape(F2, F1, K2), (2, 1, 0))

    zs = _l2a_call(ld, y0)
    zs2 = zs.reshape(K2, V1 * B, F1)
    r4 = _l2b_call(zs2, W2kT, b2.reshape(1, F2))

    A = jnp.transpose(r4.reshape(V1 // 4, B, F2), (1, 2, 0)).reshape(B, -1)

    return _p2b_call(A, Wf1, bf1.reshape(1, FC1F), Wf2, bf2.reshape(1, FC2F))


# speedup vs baseline: 16.4415x; 1.2656x over previous
import functools

import jax
import jax.numpy as jnp
from jax import lax
from jax.experimental import pallas as pl
from jax.experimental.pallas import tpu as pltpu
from jax.experimental.pallas import tpu_sc as plsc

V0 = 4096
V1 = 1024
B = 16
K1 = 5
K2 = 5
F1 = 32
F2 = 64
FC1F = 512
FC2F = 10
E0 = V0 * 16
E2 = V1 * 16

NC = 2
NS = 16
LANES = 16
CH = 128

EPW0 = E0 // (NC * NS)
NCH0 = EPW0 // CH
RPT0 = V0 // NS

EPW2 = E2 // NS
NCH2 = EPW2 // CH
DRT = (V1 * V1 // LANES) // NS


def _sc_spmm1_body(x_hbm, src_hbm, dst_hbm, ev_hbm, out_hbm,
                   idx_s, idx_d, ev_v, rows_v, acc_sh, sem, sem2):
    c = lax.axis_index("c")
    s = lax.axis_index("s")
    w = c * NS + s

    def _z(i, _):
        rows_v[i] = jnp.zeros((LANES,), jnp.float32)
        return 0
    lax.fori_loop(0, RPT0, _z, 0)
    pltpu.sync_copy(rows_v.at[pl.ds(0, RPT0)],
                    acc_sh.at[pl.ds(s * RPT0, RPT0)])

    pltpu.sync_copy(src_hbm.at[pl.ds(w * EPW0, EPW0)], idx_s)
    pltpu.sync_copy(dst_hbm.at[pl.ds(w * NCH0, NCH0)], idx_d)
    pltpu.sync_copy(ev_hbm.at[pl.ds(w * EPW0, EPW0)], ev_v)

    plsc.subcore_barrier()

    descs = []
    for j in range(NCH0):
        descs.append(pltpu.async_copy(
            x_hbm.at[idx_s.at[pl.ds(j * CH, CH)]],
            rows_v.at[pl.ds(j * CH, CH)], sem))
    sdescs = []
    for j in range(NCH0):
        descs[j].wait()

        def _m(g, _, base0=j * CH):
            base = base0 + g * LANES
            ev16 = ev_v[pl.ds(base, LANES)]
            for t in range(LANES):
                rows_v[base + t] = rows_v[base + t] * ev16[t]
            return 0
        lax.fori_loop(0, CH // LANES, _m, 0, unroll=2)
        sdescs.append(pltpu.async_copy(
            rows_v.at[pl.ds(j * CH, CH)],
            acc_sh.at[idx_d.at[j]], sem2, add=True))
    for d in sdescs:
        d.wait()

    plsc.subcore_barrier()

    pltpu.sync_copy(acc_sh.at[pl.ds(s * RPT0, RPT0)],
                    out_hbm.at[c].at[pl.ds(s * RPT0, RPT0)])


def _sc_densify_body(src_hbm, dst_hbm, ev_hbm, out_hbm,
                     idx_s, idx_d, ev_v, sidx, rows_v, ld_sh, sem):
    c = lax.axis_index("c")
    s = lax.axis_index("s")

    @pl.when(c == 0)
    def _():
        def _z(i, _):
            rows_v[i] = jnp.zeros((LANES,), jnp.float32)
            return 0
        lax.fori_loop(0, RPT0, _z, 0)
        for r in range(DRT // RPT0):
            pltpu.sync_copy(
                rows_v.at[pl.ds(0, RPT0)],
                ld_sh.at[pl.ds(s * DRT + r * RPT0, RPT0)])

        pltpu.sync_copy(src_hbm.at[pl.ds(s * EPW2, EPW2)], idx_s)
        pltpu.sync_copy(dst_hbm.at[pl.ds(s * EPW2, EPW2)], idx_d)
        pltpu.sync_copy(ev_hbm.at[pl.ds(s * EPW2, EPW2)], ev_v)

        plsc.subcore_barrier()

        lane_ids = lax.iota(jnp.int32, LANES)

        def _mk(g, _):
            base = g * LANES
            sv = idx_s[pl.ds(base, LANES)]
            dv = idx_d[pl.ds(base, LANES)]
            ev16 = ev_v[pl.ds(base, LANES)]
            srow = dv * (V1 // LANES) + (sv >> 4)
            lane = sv & (LANES - 1)
            chunk = g // (CH // LANES)
            off = (g % (CH // LANES)) * LANES
            sidx[chunk, pl.ds(off, LANES)] = srow
            for t in range(LANES):
                rows_v[base + t] = jnp.where(lane_ids == lane[t], ev16[t], 0.0)
            return 0
        lax.fori_loop(0, EPW2 // LANES, _mk, 0)

        sdescs = []
        for j in range(NCH2):
            sdescs.append(pltpu.async_copy(rows_v.at[pl.ds(j * CH, CH)],
                                           ld_sh.at[sidx.at[j]], sem,
                                           add=True))
        for d in sdescs:
            d.wait()

        plsc.subcore_barrier()

        pltpu.sync_copy(ld_sh.at[pl.ds(s * DRT, DRT)],
                        out_hbm.at[pl.ds(s * DRT, DRT)])


@functools.lru_cache(maxsize=1)
def _build_sc_kernels():
    mesh = plsc.VectorSubcoreMesh(core_axis_name="c", subcore_axis_name="s")
    params = pltpu.CompilerParams(use_tc_tiling_on_sc=False)
    spmm1 = pl.kernel(
        _sc_spmm1_body,
        out_type=jax.ShapeDtypeStruct((NC, V0, B), jnp.float32),
        mesh=mesh,
        compiler_params=params,
        scratch_types=[
            pltpu.VMEM((EPW0,), jnp.int32),
            pltpu.VMEM((NCH0, CH), jnp.int32),
            pltpu.VMEM((EPW0,), jnp.float32),
            pltpu.VMEM((EPW0, LANES), jnp.float32),
            pltpu.VMEM_SHARED((V0, B), jnp.float32),
            pltpu.SemaphoreType.DMA,
            pltpu.SemaphoreType.DMA,
        ],
    )
    densify = pl.kernel(
        _sc_densify_body,
        out_type=jax.ShapeDtypeStruct((V1 * V1 // LANES, LANES), jnp.float32),
        mesh=mesh,
        compiler_params=params,
        scratch_types=[
            pltpu.VMEM((EPW2,), jnp.int32),
            pltpu.VMEM((EPW2,), jnp.int32),
            pltpu.VMEM((EPW2,), jnp.float32),
            pltpu.VMEM((NCH2, CH), jnp.int32),
            pltpu.VMEM((EPW2, LANES), jnp.float32),
            pltpu.VMEM_SHARED((V1 * V1 // LANES, LANES), jnp.float32),
            pltpu.SemaphoreType.DMA,
        ],
    )
    return spmm1, densify


def _comb_first_body(p_ref, out_ref):
    out_ref[...] = p_ref[0] + p_ref[1]


def _comb_next_body(p_ref, prev_ref, out_ref):
    out_ref[...] = 2.0 * (p_ref[0] + p_ref[1]) - prev_ref[...]


def _p1_body(t_ref, p_ref, b_ref, out_ref):
    G = jnp.dot(t_ref[...], p_ref[...], preferred_element_type=jnp.float32)
    G = jnp.maximum(G + b_ref[...], 0.0)
    out_ref[...] = G.reshape(V1, 4, B * F1).max(axis=1)


def _l2_body(ld_ref, y0_ref, w2_ref, b2_ref, out_ref):
    Ld = ld_ref[...]
    z0 = y0_ref[...]
    hi = jax.lax.Precision.HIGHEST
    z1 = jnp.dot(Ld, z0, preferred_element_type=jnp.float32, precision=hi)
    z2 = 2.0 * jnp.dot(Ld, z1, preferred_element_type=jnp.float32,
                       precision=hi) - z0
    z3 = 2.0 * jnp.dot(Ld, z2, preferred_element_type=jnp.float32) - z1
    z4 = 2.0 * jnp.dot(Ld, z3, preferred_element_type=jnp.float32) - z2
    zs = (z0, z1, z2, z3, z4)
    b2 = b2_ref[...]
    for b in range(B):
        acc = jnp.broadcast_to(b2, (V1, F2))
        for k in range(K2):
            acc = acc + jnp.dot(zs[k][:, b * F1:(b + 1) * F1], w2_ref[k],
                                preferred_element_type=jnp.float32)
        r = jnp.maximum(acc, 0.0)
        out_ref[:, b * F2:(b + 1) * F2] = r.reshape(V1 // 4, 4, F2).max(axis=1)


def _p2b_body(a_ref, wf1_ref, bf1_ref, wf2_ref, bf2_ref, out_ref):
    h1 = jax.lax.dot_general(a_ref[...], wf1_ref[...], (((1,), (1,)), ((), ())),
                             preferred_element_type=jnp.float32)
    h1 = jnp.maximum(h1 + bf1_ref[...], 0.0)
    out = jax.lax.dot_general(h1, wf2_ref[...], (((1,), (1,)), ((), ())),
                              preferred_element_type=jnp.float32)
    out_ref[...] = out + bf2_ref[...]


_comb_first = pl.pallas_call(
    _comb_first_body,
    out_shape=jax.ShapeDtypeStruct((V0 * B // 128, 128), jnp.float32))
_comb_next = pl.pallas_call(
    _comb_next_body,
    out_shape=jax.ShapeDtypeStruct((V0 * B // 128, 128), jnp.float32))
_p1_call = pl.pallas_call(
    _p1_body, out_shape=jax.ShapeDtypeStruct((V1, B * F1), jnp.float32))
_l2_call = pl.pallas_call(
    _l2_body, out_shape=jax.ShapeDtypeStruct((V1 // 4, B * F2), jnp.float32))
_p2b_call = pl.pallas_call(
    _p2b_body, out_shape=jax.ShapeDtypeStruct((B, FC2F), jnp.float32))


def kernel(x, ei0, ev0, ei2, ev2, W1, b1, W2, b2, Wf1, bf1, Wf2, bf2):
    src0 = ei0[0]
    dst0 = ei0[1].reshape(E0 // CH, CH)
    src2 = ei2[0]
    dst2 = ei2[1]

    sc_spmm1, sc_densify = _build_sc_kernels()

    ld = sc_densify(src2, dst2, ev2).reshape(V1, V1)

    w128 = (NC, V0 * B // 128, 128)
    x0 = x[:, 0, :].T
    x0w = x0.reshape(w128[1:])
    p = sc_spmm1(x0, src0, dst0, ev0)
    x1w = _comb_first(p.reshape(w128))
    p = sc_spmm1(x1w.reshape(V0, B), src0, dst0, ev0)
    x2w = _comb_next(p.reshape(w128), x0w)
    p = sc_spmm1(x2w.reshape(V0, B), src0, dst0, ev0)
    x3w = _comb_next(p.reshape(w128), x1w)
    p = sc_spmm1(x3w.reshape(V0, B), src0, dst0, ev0)
    x4w = _comb_next(p.reshape(w128), x2w)

    xs = [x0] + [a.reshape(V0, B) for a in (x1w, x2w, x3w, x4w)]
    T = jnp.stack(xs, axis=1).reshape(V0, K1 * B)

    P = jnp.einsum('ck,bd->kbdc', W1, jnp.eye(B, dtype=W1.dtype))
    P = P.reshape(K1 * B, B * F1)
    b1big = jnp.tile(b1, B).reshape(1, B * F1)

    y0 = _p1_call(T, P, b1big)

    W2kT = jnp.transpose(W2.reshape(F2, F1, K2), (2, 1, 0))

    r4 = _l2_call(ld, y0, W2kT, b2.reshape(1, F2))

    A = jnp.transpose(r4.reshape(V1 // 4, B, F2), (1, 2, 0)).reshape(B, -1)

    return _p2b_call(A, Wf1, bf1.reshape(1, FC1F), Wf2, bf2.reshape(1, FC2F))


# speedup vs baseline: 16.6611x; 1.0134x over previous
import functools

import jax
import jax.numpy as jnp
from jax import lax
from jax.experimental import pallas as pl
from jax.experimental.pallas import tpu as pltpu
from jax.experimental.pallas import tpu_sc as plsc

V0 = 4096
V1 = 1024
B = 16
K1 = 5
K2 = 5
F1 = 32
F2 = 64
FC1F = 512
FC2F = 10
E0 = V0 * 16
E2 = V1 * 16

NC = 2
NS = 16
LANES = 16
CH = 128

EPW0 = E0 // (NC * NS)
NCH0 = EPW0 // CH
RPT0 = V0 // NS

EPW2 = E2 // NS
NCH2 = EPW2 // CH
DRT = (V1 * V1 // LANES) // NS


def _sc_spmm1_body(x_hbm, src_hbm, dst_hbm, ev_hbm, out_hbm,
                   idx_s, idx_d, ev_v, rows_v, zbuf, acc_sh, sem, sem2):
    c = lax.axis_index("c")
    s = lax.axis_index("s")
    w = c * NS + s

    pltpu.sync_copy(src_hbm.at[pl.ds(w * EPW0, EPW0)], idx_s)
    pltpu.sync_copy(dst_hbm.at[pl.ds(w * NCH0, NCH0)], idx_d)
    pltpu.sync_copy(ev_hbm.at[pl.ds(w * EPW0, EPW0)], ev_v)

    descs = []
    for j in range(NCH0):
        descs.append(pltpu.async_copy(
            x_hbm.at[idx_s.at[pl.ds(j * CH, CH)]],
            rows_v.at[pl.ds(j * CH, CH)], sem))

    def _z(i, _):
        zbuf[i] = jnp.zeros((LANES,), jnp.float32)
        return 0
    lax.fori_loop(0, RPT0, _z, 0, unroll=4)
    pltpu.sync_copy(zbuf, acc_sh.at[pl.ds(s * RPT0, RPT0)])

    plsc.subcore_barrier()
    sdescs = []
    for j in range(NCH0):
        descs[j].wait()

        def _m(g, _, base0=j * CH):
            base = base0 + g * LANES
            ev16 = ev_v[pl.ds(base, LANES)]
            for t in range(LANES):
                rows_v[base + t] = rows_v[base + t] * ev16[t]
            return 0
        lax.fori_loop(0, CH // LANES, _m, 0, unroll=4)
        sdescs.append(pltpu.async_copy(
            rows_v.at[pl.ds(j * CH, CH)],
            acc_sh.at[idx_d.at[j]], sem2, add=True))
    for d in sdescs:
        d.wait()

    plsc.subcore_barrier()

    pltpu.sync_copy(acc_sh.at[pl.ds(s * RPT0, RPT0)],
                    out_hbm.at[c].at[pl.ds(s * RPT0, RPT0)])


def _sc_densify_body(src_hbm, dst_hbm, ev_hbm, out_hbm,
                     idx_s, idx_d, ev_v, sidx, rows_v, ld_sh, sem):
    c = lax.axis_index("c")
    s = lax.axis_index("s")

    @pl.when(c == 0)
    def _():
        def _z(i, _):
            rows_v[i] = jnp.zeros((LANES,), jnp.float32)
            return 0
        lax.fori_loop(0, RPT0, _z, 0)
        for r in range(DRT // RPT0):
            pltpu.sync_copy(
                rows_v.at[pl.ds(0, RPT0)],
                ld_sh.at[pl.ds(s * DRT + r * RPT0, RPT0)])

        pltpu.sync_copy(src_hbm.at[pl.ds(s * EPW2, EPW2)], idx_s)
        pltpu.sync_copy(dst_hbm.at[pl.ds(s * EPW2, EPW2)], idx_d)
        pltpu.sync_copy(ev_hbm.at[pl.ds(s * EPW2, EPW2)], ev_v)

        plsc.subcore_barrier()

        lane_ids = lax.iota(jnp.int32, LANES)

        def _mk(g, _):
            base = g * LANES
            sv = idx_s[pl.ds(base, LANES)]
            dv = idx_d[pl.ds(base, LANES)]
            ev16 = ev_v[pl.ds(base, LANES)]
            srow = dv * (V1 // LANES) + (sv >> 4)
            lane = sv & (LANES - 1)
            chunk = g // (CH // LANES)
            off = (g % (CH // LANES)) * LANES
            sidx[chunk, pl.ds(off, LANES)] = srow
            for t in range(LANES):
                rows_v[base + t] = jnp.where(lane_ids == lane[t], ev16[t], 0.0)
            return 0
        lax.fori_loop(0, EPW2 // LANES, _mk, 0)

        sdescs = []
        for j in range(NCH2):
            sdescs.append(pltpu.async_copy(rows_v.at[pl.ds(j * CH, CH)],
                                           ld_sh.at[sidx.at[j]], sem,
                                           add=True))
        for d in sdescs:
            d.wait()

        plsc.subcore_barrier()

        pltpu.sync_copy(ld_sh.at[pl.ds(s * DRT, DRT)],
                        out_hbm.at[pl.ds(s * DRT, DRT)])


@functools.lru_cache(maxsize=1)
def _build_sc_kernels():
    mesh = plsc.VectorSubcoreMesh(core_axis_name="c", subcore_axis_name="s")
    params = pltpu.CompilerParams(use_tc_tiling_on_sc=False)
    spmm1 = pl.kernel(
        _sc_spmm1_body,
        out_type=jax.ShapeDtypeStruct((NC, V0, B), jnp.float32),
        mesh=mesh,
        compiler_params=params,
        scratch_types=[
            pltpu.VMEM((EPW0,), jnp.int32),
            pltpu.VMEM((NCH0, CH), jnp.int32),
            pltpu.VMEM((EPW0,), jnp.float32),
            pltpu.VMEM((EPW0, LANES), jnp.float32),
            pltpu.VMEM((RPT0, LANES), jnp.float32),
            pltpu.VMEM_SHARED((V0, B), jnp.float32),
            pltpu.SemaphoreType.DMA,
            pltpu.SemaphoreType.DMA,
        ],
    )
    densify = pl.kernel(
        _sc_densify_body,
        out_type=jax.ShapeDtypeStruct((V1 * V1 // LANES, LANES), jnp.float32),
        mesh=mesh,
        compiler_params=params,
        scratch_types=[
            pltpu.VMEM((EPW2,), jnp.int32),
            pltpu.VMEM((EPW2,), jnp.int32),
            pltpu.VMEM((EPW2,), jnp.float32),
            pltpu.VMEM((NCH2, CH), jnp.int32),
            pltpu.VMEM((EPW2, LANES), jnp.float32),
            pltpu.VMEM_SHARED((V1 * V1 // LANES, LANES), jnp.float32),
            pltpu.SemaphoreType.DMA,
        ],
    )
    return spmm1, densify


def _comb_first_body(p_ref, out_ref):
    out_ref[...] = p_ref[0] + p_ref[1]


def _comb_next_body(p_ref, prev_ref, out_ref):
    out_ref[...] = 2.0 * (p_ref[0] + p_ref[1]) - prev_ref[...]


def _p1_body(t_ref, p_ref, b_ref, out_ref):
    G = jnp.dot(t_ref[...], p_ref[...], preferred_element_type=jnp.float32)
    G = jnp.maximum(G + b_ref[...], 0.0)
    out_ref[...] = G.reshape(V1, 4, B * F1).max(axis=1)


def _l2_body(ld_ref, y0_ref, w2_ref, b2_ref, out_ref):
    Ld = ld_ref[...]
    z0 = y0_ref[...]
    hi = jax.lax.Precision.HIGHEST
    z1 = jnp.dot(Ld, z0, preferred_element_type=jnp.float32, precision=hi)
    z2 = 2.0 * jnp.dot(Ld, z1, preferred_element_type=jnp.float32,
                       precision=hi) - z0
    z3 = 2.0 * jnp.dot(Ld, z2, preferred_element_type=jnp.float32) - z1
    z4 = 2.0 * jnp.dot(Ld, z3, preferred_element_type=jnp.float32) - z2
    zs = (z0, z1, z2, z3, z4)
    b2 = b2_ref[...]
    for b in range(B):
        acc = jnp.broadcast_to(b2, (V1, F2))
        for k in range(K2):
            acc = acc + jnp.dot(zs[k][:, b * F1:(b + 1) * F1], w2_ref[k],
                                preferred_element_type=jnp.float32)
        r = jnp.maximum(acc, 0.0)
        out_ref[:, b * F2:(b + 1) * F2] = r.reshape(V1 // 4, 4, F2).max(axis=1)


def _p2b_body(a_ref, wf1_ref, bf1_ref, wf2_ref, bf2_ref, out_ref):
    h1 = jax.lax.dot_general(a_ref[...], wf1_ref[...], (((1,), (1,)), ((), ())),
                             preferred_element_type=jnp.float32)
    h1 = jnp.maximum(h1 + bf1_ref[...], 0.0)
    out = jax.lax.dot_general(h1, wf2_ref[...], (((1,), (1,)), ((), ())),
                              preferred_element_type=jnp.float32)
    out_ref[...] = out + bf2_ref[...]


_comb_first = pl.pallas_call(
    _comb_first_body,
    out_shape=jax.ShapeDtypeStruct((V0 * B // 128, 128), jnp.float32))
_comb_next = pl.pallas_call(
    _comb_next_body,
    out_shape=jax.ShapeDtypeStruct((V0 * B // 128, 128), jnp.float32))
_p1_call = pl.pallas_call(
    _p1_body, out_shape=jax.ShapeDtypeStruct((V1, B * F1), jnp.float32))
_l2_call = pl.pallas_call(
    _l2_body, out_shape=jax.ShapeDtypeStruct((V1 // 4, B * F2), jnp.float32))
_p2b_call = pl.pallas_call(
    _p2b_body, out_shape=jax.ShapeDtypeStruct((B, FC2F), jnp.float32))


def kernel(x, ei0, ev0, ei2, ev2, W1, b1, W2, b2, Wf1, bf1, Wf2, bf2):
    src0 = ei0[0]
    dst0 = ei0[1].reshape(E0 // CH, CH)
    src2 = ei2[0]
    dst2 = ei2[1]

    sc_spmm1, sc_densify = _build_sc_kernels()

    ld = sc_densify(src2, dst2, ev2).reshape(V1, V1)

    w128 = (NC, V0 * B // 128, 128)
    x0 = x[:, 0, :].T
    x0w = x0.reshape(w128[1:])
    p = sc_spmm1(x0, src0, dst0, ev0)
    x1w = _comb_first(p.reshape(w128))
    p = sc_spmm1(x1w.reshape(V0, B), src0, dst0, ev0)
    x2w = _comb_next(p.reshape(w128), x0w)
    p = sc_spmm1(x2w.reshape(V0, B), src0, dst0, ev0)
    x3w = _comb_next(p.reshape(w128), x1w)
    p = sc_spmm1(x3w.reshape(V0, B), src0, dst0, ev0)
    x4w = _comb_next(p.reshape(w128), x2w)

    xs = [x0] + [a.reshape(V0, B) for a in (x1w, x2w, x3w, x4w)]
    T = jnp.stack(xs, axis=1).reshape(V0, K1 * B)

    P = jnp.einsum('ck,bd->kbdc', W1, jnp.eye(B, dtype=W1.dtype))
    P = P.reshape(K1 * B, B * F1)
    b1big = jnp.tile(b1, B).reshape(1, B * F1)

    y0 = _p1_call(T, P, b1big)

    W2kT = jnp.transpose(W2.reshape(F2, F1, K2), (2, 1, 0))

    r4 = _l2_call(ld, y0, W2kT, b2.reshape(1, F2))

    A = jnp.transpose(r4.reshape(V1 // 4, B, F2), (1, 2, 0)).reshape(B, -1)

    return _p2b_call(A, Wf1, bf1.reshape(1, FC1F), Wf2, bf2.reshape(1, FC2F))
